# Initial kernel scaffold; baseline (speedup 1.0000x reference)
import jax
import jax.numpy as jnp
from jax.experimental import pallas as pl


def kernel(x, W_in_w, W_in_b, Wu_w, Wu_b, W_out_w, W_out_b, ln1_g, ln1_b, ffn1_w, ffn1_b, ffn2_w, ffn2_b, ln2_g, ln2_b):
    raise NotImplementedError("write your pallas kernel here")


# speedup vs baseline: 1.8628x; 1.8628x over previous
import functools

import jax
import jax.numpy as jnp
from jax import lax
from jax.experimental import pallas as pl
from jax.experimental.pallas import tpu as pltpu
from jax.experimental.pallas import tpu_sc as plsc

N = 10000
D = 128
K = 16
NPAD = 10240
QB = 200
NBLK = N // QB

NW = 32
TPW = 313
NROW = NW * TPW
E = N * K
CH = 2000
NCH = E // CH
GB = 128
SENT = -3.0e38


def _knn_body(xq_ref, xall_ref, sq_ref, ww_ref, wb_ref, nbr_ref, h_ref, d_ref):
    b = pl.program_id(0)
    xq = xq_ref[...]
    p = lax.dot_general(xq, xall_ref[...], (((1,), (1,)), ((), ())),
                        preferred_element_type=jnp.float32)
    d = sq_ref[...] - 2.0 * p
    gi = b * QB + lax.broadcasted_iota(jnp.int32, (QB, 1), 0)
    col = lax.broadcasted_iota(jnp.int32, (QB, NPAD), 1)
    d = jnp.where((col == gi) | (col >= N), jnp.inf, d)
    d_ref[...] = d
    idxs = []
    for _ in range(K):
        dcur = d_ref[...]
        m = jnp.min(dcur, axis=1, keepdims=True)
        isel = jnp.min(jnp.where(dcur == m, col, NPAD), axis=1, keepdims=True)
        idxs.append(isel)
        d_ref[...] = jnp.where(col == isel, jnp.inf, dcur)
    nbr_ref[...] = jnp.concatenate(idxs, axis=1)
    h_ref[...] = (jnp.dot(xq, ww_ref[...], preferred_element_type=jnp.float32)
                  + wb_ref[...])


def _knn_call(x, xpad, sq, ww, wb):
    return pl.pallas_call(
        _knn_body,
        grid=(NBLK,),
        in_specs=[
            pl.BlockSpec((QB, D), lambda b: (b, 0)),
            pl.BlockSpec((NPAD, D), lambda b: (0, 0)),
            pl.BlockSpec((1, NPAD), lambda b: (0, 0)),
            pl.BlockSpec((D, D), lambda b: (0, 0)),
            pl.BlockSpec((1, D), lambda b: (0, 0)),
        ],
        out_specs=[
            pl.BlockSpec((QB, K), lambda b: (b, 0)),
            pl.BlockSpec((QB, D), lambda b: (b, 0)),
        ],
        out_shape=[
            jax.ShapeDtypeStruct((N, K), jnp.int32),
            jax.ShapeDtypeStruct((N, D), jnp.float32),
        ],
        scratch_shapes=[pltpu.VMEM((QB, NPAD), jnp.float32)],
    )(x, xpad, sq, ww, wb)


def _scatter_body(nbr_hbm, h_hbm, out_hbm, dstc, cdst, csrc, rows, sloc, sem):
    wid = lax.axis_index("s") * 2 + lax.axis_index("c")
    lo = wid * TPW

    def init_body(i, _):
        sloc[pl.ds(i * 16, 16)] = jnp.full((16,), SENT, jnp.float32)
        return 0

    lax.fori_loop(0, (TPW + 1) * D // 16, init_body, 0)

    def init_idx(i, _):
        cdst[pl.ds(i * 16, 16)] = jnp.zeros((16,), jnp.int32)
        csrc[pl.ds(i * 16, 16)] = jnp.zeros((16,), jnp.int32)
        return 0

    lax.fori_loop(0, (CH + 16) // 16, init_idx, 0)

    def chunk_body(c, _):
        pltpu.sync_copy(nbr_hbm.at[pl.ds(c * CH, CH)], dstc)

        def scan_body(g, pos):
            dv = dstc[pl.ds(g * 16, 16)]
            eid = c * CH + g * 16 + lax.iota(jnp.int32, 16)
            srcv = lax.shift_right_logical(eid, 4)
            dl = dv - lo
            msk = (dl >= 0) & (dl < TPW)
            cum = plsc.cumsum(jnp.where(msk, 1, 0))
            lane = lax.iota(jnp.int32, 16)
            idx = jnp.where(msk, pos + cum - 1, CH + lane)
            plsc.store_scatter(cdst, [idx], dl)
            plsc.store_scatter(csrc, [idx], srcv)
            return pos + jnp.max(cum)

        mcnt = lax.fori_loop(0, CH // 16, scan_body, 0)

        def sub_body(sb, _):
            sbase = sb * GB
            pltpu.async_copy(h_hbm.at[csrc.at[pl.ds(sbase, GB)]], rows,
                             sem).wait()

            def fold_g(g, _):
                gb = sbase + g * 16
                dlv = cdst[pl.ds(gb, 16)]
                valid = (gb + lax.iota(jnp.int32, 16)) < mcnt
                jv = jnp.where(valid, dlv, TPW)
                for l in range(16):
                    j = jv[l]
                    for r in range(D // 16):
                        slot = j * D + r * 16
                        v = rows[g * 16 + l, pl.ds(r * 16, 16)]
                        sloc[pl.ds(slot, 16)] = jnp.maximum(
                            sloc[pl.ds(slot, 16)], v)
                return 0

            lax.fori_loop(0, GB // 16, fold_g, 0)
            return 0

        nsub = (mcnt + GB - 1) // GB
        lax.fori_loop(0, nsub, sub_body, 0)
        return 0

    lax.fori_loop(0, NCH, chunk_body, 0)
    pltpu.sync_copy(sloc.at[pl.ds(0, TPW * D)],
                    out_hbm.at[pl.ds(lo * D, TPW * D)])


def _scatter_call(nbr_flat, h):
    mesh = plsc.VectorSubcoreMesh(core_axis_name="c", subcore_axis_name="s")
    fn = functools.partial(
        pl.kernel, _scatter_body, mesh=mesh,
        compiler_params=pltpu.CompilerParams(needs_layout_passes=False),
        out_type=jax.ShapeDtypeStruct((NROW * D,), jnp.float32),
        scratch_types=[
            pltpu.VMEM((CH,), jnp.int32),
            pltpu.VMEM((CH + 16,), jnp.int32),
            pltpu.VMEM((CH + 16,), jnp.int32),
            pltpu.VMEM((GB, D), jnp.float32),
            pltpu.VMEM(((TPW + 1) * D,), jnp.float32),
            pltpu.SemaphoreType.DMA,
        ],
    )()
    return fn(nbr_flat, h)


RB = 1000
NRB = N // RB


def _ln(v, g, b):
    mu = jnp.mean(v, axis=-1, keepdims=True)
    var = jnp.mean((v - mu) ** 2, axis=-1, keepdims=True)
    return (v - mu) / jnp.sqrt(var + 1e-5) * g + b


def _tail_body(x_ref, h_ref, s_ref, wu_ref, wub_ref, wo_ref, wob_ref,
               g1_ref, b1_ref, f1w_ref, f1b_ref, f2w_ref, f2b_ref,
               g2_ref, b2_ref, out_ref):
    h = h_ref[...]
    s = s_ref[...]
    md = jnp.where(s > -1.0e38, s - h, 0.0)
    wu = wu_ref[...]
    u = (jnp.dot(h, wu[:D], preferred_element_type=jnp.float32)
         + jnp.dot(md, wu[D:], preferred_element_type=jnp.float32)
         + wub_ref[...])
    h2 = jnp.dot(u, wo_ref[...], preferred_element_type=jnp.float32) + wob_ref[...]
    x1 = _ln(x_ref[...] + h2, g1_ref[...], b1_ref[...])
    z = jnp.dot(x1, f1w_ref[...], preferred_element_type=jnp.float32) + f1b_ref[...]
    f = 0.5 * z * (1.0 + lax.erf(z * 0.7071067811865476))
    f = jnp.dot(f, f2w_ref[...], preferred_element_type=jnp.float32) + f2b_ref[...]
    out_ref[...] = _ln(x1 + f, g2_ref[...], b2_ref[...])


def _tail_call(x, h, s, wu, wub, wo, wob, g1, b1, f1w, f1b, f2w, f2b, g2, b2):
    row = lambda b: (b, 0)
    full = lambda b: (0, 0)
    return pl.pallas_call(
        _tail_body,
        grid=(NRB,),
        in_specs=[
            pl.BlockSpec((RB, D), row),
            pl.BlockSpec((RB, D), row),
            pl.BlockSpec((RB, D), row),
            pl.BlockSpec((2 * D, D), full),
            pl.BlockSpec((1, D), full),
            pl.BlockSpec((D, D), full),
            pl.BlockSpec((1, D), full),
            pl.BlockSpec((1, D), full),
            pl.BlockSpec((1, D), full),
            pl.BlockSpec((D, 4 * D), full),
            pl.BlockSpec((1, 4 * D), full),
            pl.BlockSpec((4 * D, D), full),
            pl.BlockSpec((1, D), full),
            pl.BlockSpec((1, D), full),
            pl.BlockSpec((1, D), full),
        ],
        out_specs=pl.BlockSpec((RB, D), row),
        out_shape=jax.ShapeDtypeStruct((N, D), jnp.float32),
    )(x, h, s, wu, wub, wo, wob, g1, b1, f1w, f1b, f2w, f2b, g2, b2)


def kernel(x, W_in_w, W_in_b, Wu_w, Wu_b, W_out_w, W_out_b, ln1_g, ln1_b,
           ffn1_w, ffn1_b, ffn2_w, ffn2_b, ln2_g, ln2_b):
    xpad = jnp.concatenate([x, jnp.zeros((NPAD - N, D), jnp.float32)], axis=0)
    sq = jnp.sum(xpad * xpad, axis=1).reshape(1, NPAD)
    nbr, h = _knn_call(x, xpad, sq, W_in_w, W_in_b.reshape(1, D))
    s_flat = _scatter_call(nbr.reshape(-1), h)
    s = s_flat.reshape(NROW, D)[:N]
    return _tail_call(x, h, s, Wu_w, Wu_b.reshape(1, D), W_out_w,
                      W_out_b.reshape(1, D), ln1_g.reshape(1, D),
                      ln1_b.reshape(1, D), ffn1_w, ffn1_b.reshape(1, 4 * D),
                      ffn2_w, ffn2_b.reshape(1, D), ln2_g.reshape(1, D),
                      ln2_b.reshape(1, D))


# speedup vs baseline: 3.7392x; 2.0073x over previous
import functools

import jax
import jax.numpy as jnp
from jax import lax
from jax.experimental import pallas as pl
from jax.experimental.pallas import tpu as pltpu
from jax.experimental.pallas import tpu_sc as plsc

N = 10000
D = 128
K = 16
NPAD = 10240
QB = 200
NBLK = N // QB

NW = 32
TPW = 313
NROW = NW * TPW
E = N * K
CH = 3200
NCH = E // CH
GB = 128
SENT = -3.0e38


def _knn_body(xq_ref, xall_ref, sq_ref, ww_ref, wb_ref, nbr_ref, h_ref, d_ref):
    b = pl.program_id(0)
    xq = xq_ref[...]
    p = lax.dot_general(xq, xall_ref[...], (((1,), (1,)), ((), ())),
                        preferred_element_type=jnp.float32)
    d = sq_ref[...] - 2.0 * p
    gi = b * QB + lax.broadcasted_iota(jnp.int32, (QB, 1), 0)
    col = lax.broadcasted_iota(jnp.int32, (QB, NPAD), 1)
    d = jnp.where((col == gi) | (col >= N), jnp.inf, d)
    d_ref[...] = d
    idxs = []
    for _ in range(K):
        dcur = d_ref[...]
        m = jnp.min(dcur, axis=1, keepdims=True)
        isel = jnp.min(jnp.where(dcur == m, col, NPAD), axis=1, keepdims=True)
        idxs.append(isel)
        d_ref[...] = jnp.where(col == isel, jnp.inf, dcur)
    nbr_ref[...] = jnp.concatenate(idxs, axis=1)
    h_ref[...] = (jnp.dot(xq, ww_ref[...], preferred_element_type=jnp.float32)
                  + wb_ref[...])


def _knn_call(x, xpad, sq, ww, wb):
    return pl.pallas_call(
        _knn_body,
        grid=(NBLK,),
        in_specs=[
            pl.BlockSpec((QB, D), lambda b: (b, 0)),
            pl.BlockSpec((NPAD, D), lambda b: (0, 0)),
            pl.BlockSpec((1, NPAD), lambda b: (0, 0)),
            pl.BlockSpec((D, D), lambda b: (0, 0)),
            pl.BlockSpec((1, D), lambda b: (0, 0)),
        ],
        out_specs=[
            pl.BlockSpec((QB, K), lambda b: (b, 0)),
            pl.BlockSpec((QB, D), lambda b: (b, 0)),
        ],
        out_shape=[
            jax.ShapeDtypeStruct((N, K), jnp.int32),
            jax.ShapeDtypeStruct((N, D), jnp.float32),
        ],
        scratch_shapes=[pltpu.VMEM((QB, NPAD), jnp.float32)],
    )(x, xpad, sq, ww, wb)


def _scatter_body(nbr_hbm, h_hbm, out_hbm, d0, d1, c0, c1, s0, s1, r0, r1,
                  sloc, sd0, sd1, sg0, sg1):
    dstc = (d0, d1)
    cdst = (c0, c1)
    csrc = (s0, s1)
    rows = (r0, r1)
    sd = (sd0, sd1)
    sg = (sg0, sg1)
    wid = lax.axis_index("s") * 2 + lax.axis_index("c")
    lo = wid * TPW

    def init_body(i, _):
        sloc[pl.ds(i * 16, 16)] = jnp.full((16,), SENT, jnp.float32)
        return 0

    lax.fori_loop(0, (TPW + 1) * D // 16, init_body, 0)

    def init_idx(i, _):
        z = jnp.zeros((16,), jnp.int32)
        c0[pl.ds(i * 16, 16)] = z
        c1[pl.ds(i * 16, 16)] = z
        s0[pl.ds(i * 16, 16)] = z
        s1[pl.ds(i * 16, 16)] = z
        return 0

    lax.fori_loop(0, (CH + 16) // 16, init_idx, 0)

    pltpu.async_copy(nbr_hbm.at[pl.ds(0, CH)], d0, sd0)
    pltpu.async_copy(h_hbm.at[s1.at[pl.ds(0, GB)]], r1, sg1)

    def scan(p, c):
        pltpu.make_async_copy(nbr_hbm.at[pl.ds(c * CH, CH)], dstc[p],
                              sd[p]).wait()

        @pl.when(c + 1 < NCH)
        def _():
            pltpu.async_copy(nbr_hbm.at[pl.ds((c + 1) * CH, CH)],
                             dstc[1 - p], sd[1 - p])

        lane = lax.iota(jnp.int32, 16)

        def scan_body(g, pos):
            dv = dstc[p][pl.ds(g * 16, 16)]
            eid = c * CH + g * 16 + lane
            srcv = lax.shift_right_logical(eid, 4)
            dl = dv - lo
            msk = (dl >= 0) & (dl < TPW)
            cum = plsc.cumsum(jnp.where(msk, 1, 0))
            idx = jnp.where(msk, pos + cum - 1, CH + lane)
            plsc.store_scatter(cdst[p], [idx], dl)
            plsc.store_scatter(csrc[p], [idx], srcv)
            return pos + cum[15]

        mcnt = lax.fori_loop(0, CH // 16, scan_body, 0)
        pltpu.async_copy(h_hbm.at[csrc[p].at[pl.ds(0, GB)]], rows[p], sg[p])
        return mcnt

    def fold(p, mcnt):
        lane = lax.iota(jnp.int32, 16)
        nsub = jnp.maximum((mcnt + GB - 1) // GB, 1)

        def sub_body(sb, _):
            sbase = sb * GB

            @pl.when(sb > 0)
            def _():
                pltpu.async_copy(h_hbm.at[csrc[p].at[pl.ds(sbase, GB)]],
                                 rows[p], sg[p])

            pltpu.make_async_copy(h_hbm.at[csrc[p].at[pl.ds(sbase, GB)]],
                                  rows[p], sg[p]).wait()
            nrem = jnp.maximum(jnp.minimum(mcnt - sbase, GB), 0)

            def fold_g(g, _):
                gb = g * 16
                dlv = cdst[p][pl.ds(sbase + gb, 16)]
                valid = (gb + lane) < nrem
                jv = jnp.where(valid, dlv, TPW)
                for l in range(16):
                    j = jv[l]
                    for r in range(D // 16):
                        slot = j * D + r * 16
                        v = rows[p][gb + l, pl.ds(r * 16, 16)]
                        sloc[pl.ds(slot, 16)] = jnp.maximum(
                            sloc[pl.ds(slot, 16)], v)
                return 0

            lax.fori_loop(0, (nrem + 15) // 16, fold_g, 0)
            return 0

        lax.fori_loop(0, nsub, sub_body, 0)

    def body(i, mprev):
        a = 2 * i
        ma = scan(0, a)
        fold(1, mprev)
        mb = scan(1, a + 1)
        fold(0, ma)
        return mb

    mlast = lax.fori_loop(0, NCH // 2, body, 0)
    fold(1, mlast)
    pltpu.sync_copy(sloc.at[pl.ds(0, TPW * D)],
                    out_hbm.at[pl.ds(lo * D, TPW * D)])


def _scatter_call(nbr_flat, h):
    mesh = plsc.VectorSubcoreMesh(core_axis_name="c", subcore_axis_name="s")
    fn = functools.partial(
        pl.kernel, _scatter_body, mesh=mesh,
        compiler_params=pltpu.CompilerParams(needs_layout_passes=False),
        out_type=jax.ShapeDtypeStruct((NROW * D,), jnp.float32),
        scratch_types=[
            pltpu.VMEM((CH,), jnp.int32),
            pltpu.VMEM((CH,), jnp.int32),
            pltpu.VMEM((CH + 16,), jnp.int32),
            pltpu.VMEM((CH + 16,), jnp.int32),
            pltpu.VMEM((CH + 16,), jnp.int32),
            pltpu.VMEM((CH + 16,), jnp.int32),
            pltpu.VMEM((GB, D), jnp.float32),
            pltpu.VMEM((GB, D), jnp.float32),
            pltpu.VMEM(((TPW + 1) * D,), jnp.float32),
            pltpu.SemaphoreType.DMA,
            pltpu.SemaphoreType.DMA,
            pltpu.SemaphoreType.DMA,
            pltpu.SemaphoreType.DMA,
        ],
    )()
    return fn(nbr_flat, h)


RB = 1000
NRB = N // RB


def _ln(v, g, b):
    mu = jnp.mean(v, axis=-1, keepdims=True)
    var = jnp.mean((v - mu) ** 2, axis=-1, keepdims=True)
    return (v - mu) / jnp.sqrt(var + 1e-5) * g + b


def _tail_body(x_ref, h_ref, s_ref, wu_ref, wub_ref, wo_ref, wob_ref,
               g1_ref, b1_ref, f1w_ref, f1b_ref, f2w_ref, f2b_ref,
               g2_ref, b2_ref, out_ref):
    h = h_ref[...]
    s = s_ref[...]
    md = jnp.where(s > -1.0e38, s - h, 0.0)
    wu = wu_ref[...]
    u = (jnp.dot(h, wu[:D], preferred_element_type=jnp.float32)
         + jnp.dot(md, wu[D:], preferred_element_type=jnp.float32)
         + wub_ref[...])
    h2 = jnp.dot(u, wo_ref[...], preferred_element_type=jnp.float32) + wob_ref[...]
    x1 = _ln(x_ref[...] + h2, g1_ref[...], b1_ref[...])
    z = jnp.dot(x1, f1w_ref[...], preferred_element_type=jnp.float32) + f1b_ref[...]
    f = 0.5 * z * (1.0 + lax.erf(z * 0.7071067811865476))
    f = jnp.dot(f, f2w_ref[...], preferred_element_type=jnp.float32) + f2b_ref[...]
    out_ref[...] = _ln(x1 + f, g2_ref[...], b2_ref[...])


def _tail_call(x, h, s, wu, wub, wo, wob, g1, b1, f1w, f1b, f2w, f2b, g2, b2):
    row = lambda b: (b, 0)
    full = lambda b: (0, 0)
    return pl.pallas_call(
        _tail_body,
        grid=(NRB,),
        in_specs=[
            pl.BlockSpec((RB, D), row),
            pl.BlockSpec((RB, D), row),
            pl.BlockSpec((RB, D), row),
            pl.BlockSpec((2 * D, D), full),
            pl.BlockSpec((1, D), full),
            pl.BlockSpec((D, D), full),
            pl.BlockSpec((1, D), full),
            pl.BlockSpec((1, D), full),
            pl.BlockSpec((1, D), full),
            pl.BlockSpec((D, 4 * D), full),
            pl.BlockSpec((1, 4 * D), full),
            pl.BlockSpec((4 * D, D), full),
            pl.BlockSpec((1, D), full),
            pl.BlockSpec((1, D), full),
            pl.BlockSpec((1, D), full),
        ],
        out_specs=pl.BlockSpec((RB, D), row),
        out_shape=jax.ShapeDtypeStruct((N, D), jnp.float32),
    )(x, h, s, wu, wub, wo, wob, g1, b1, f1w, f1b, f2w, f2b, g2, b2)


def kernel(x, W_in_w, W_in_b, Wu_w, Wu_b, W_out_w, W_out_b, ln1_g, ln1_b,
           ffn1_w, ffn1_b, ffn2_w, ffn2_b, ln2_g, ln2_b):
    xpad = jnp.concatenate([x, jnp.zeros((NPAD - N, D), jnp.float32)], axis=0)
    sq = jnp.sum(xpad * xpad, axis=1).reshape(1, NPAD)
    nbr, h = _knn_call(x, xpad, sq, W_in_w, W_in_b.reshape(1, D))
    s_flat = _scatter_call(nbr.reshape(-1), h)
    s = s_flat.reshape(NROW, D)[:N]
    return _tail_call(x, h, s, Wu_w, Wu_b.reshape(1, D), W_out_w,
                      W_out_b.reshape(1, D), ln1_g.reshape(1, D),
                      ln1_b.reshape(1, D), ffn1_w, ffn1_b.reshape(1, 4 * D),
                      ffn2_w, ffn2_b.reshape(1, D), ln2_g.reshape(1, D),
                      ln2_b.reshape(1, D))


# speedup vs baseline: 3.7791x; 1.0107x over previous
import functools

import jax
import jax.numpy as jnp
from jax import lax
from jax.experimental import pallas as pl
from jax.experimental.pallas import tpu as pltpu
from jax.experimental.pallas import tpu_sc as plsc

N = 10000
D = 128
K = 16
NPAD = 10240
QB = 200
NBLK = N // QB

NW = 32
TPW = 313
NROW = NW * TPW
E = N * K
CH = 3200
NCH = E // CH
GB = 128
SENT = -3.0e38


def _knn_body(xq_ref, xall_ref, sq_ref, ww_ref, wb_ref, nbr_ref, h_ref, d_ref):
    b = pl.program_id(0)
    xq = xq_ref[...]
    p = lax.dot_general(xq, xall_ref[...], (((1,), (1,)), ((), ())),
                        preferred_element_type=jnp.float32)
    d = sq_ref[...] - 2.0 * p
    gi = b * QB + lax.broadcasted_iota(jnp.int32, (QB, 1), 0)
    col = lax.broadcasted_iota(jnp.int32, (QB, NPAD), 1)
    d = jnp.where((col == gi) | (col >= N), jnp.inf, d)
    d_ref[...] = d
    idxs = []
    for _ in range(K):
        dcur = d_ref[...]
        m = jnp.min(dcur, axis=1, keepdims=True)
        isel = jnp.min(jnp.where(dcur == m, col, NPAD), axis=1, keepdims=True)
        idxs.append(isel)
        d_ref[...] = jnp.where(col == isel, jnp.inf, dcur)
    nbr_ref[...] = jnp.concatenate(idxs, axis=1)
    h_ref[...] = (jnp.dot(xq, ww_ref[...], preferred_element_type=jnp.float32)
                  + wb_ref[...])


def _knn_call(x, xpad, sq, ww, wb):
    return pl.pallas_call(
        _knn_body,
        grid=(NBLK,),
        in_specs=[
            pl.BlockSpec((QB, D), lambda b: (b, 0)),
            pl.BlockSpec((NPAD, D), lambda b: (0, 0)),
            pl.BlockSpec((1, NPAD), lambda b: (0, 0)),
            pl.BlockSpec((D, D), lambda b: (0, 0)),
            pl.BlockSpec((1, D), lambda b: (0, 0)),
        ],
        out_specs=[
            pl.BlockSpec((QB, K), lambda b: (b, 0)),
            pl.BlockSpec((QB, D), lambda b: (b, 0)),
        ],
        out_shape=[
            jax.ShapeDtypeStruct((N, K), jnp.int32),
            jax.ShapeDtypeStruct((N, D), jnp.float32),
        ],
        scratch_shapes=[pltpu.VMEM((QB, NPAD), jnp.float32)],
    )(x, xpad, sq, ww, wb)


def _scatter_body(nbr_hbm, h_hbm, out_hbm, d0, d1, c0, c1, s0, s1, r0, r1,
                  sloc, sd0, sd1, sg0, sg1):
    dstc = (d0, d1)
    cdst = (c0, c1)
    csrc = (s0, s1)
    rows = (r0, r1)
    sd = (sd0, sd1)
    sg = (sg0, sg1)
    wid = lax.axis_index("s") * 2 + lax.axis_index("c")
    lo = wid * TPW

    def init_body(i, _):
        sloc[pl.ds(i * 16, 16)] = jnp.full((16,), SENT, jnp.float32)
        return 0

    lax.fori_loop(0, (TPW + 1) * D // 16, init_body, 0)

    def init_idx(i, _):
        z = jnp.zeros((16,), jnp.int32)
        c0[pl.ds(i * 16, 16)] = z
        c1[pl.ds(i * 16, 16)] = z
        s0[pl.ds(i * 16, 16)] = z
        s1[pl.ds(i * 16, 16)] = z
        return 0

    lax.fori_loop(0, (CH + 16) // 16, init_idx, 0)

    pltpu.async_copy(nbr_hbm.at[pl.ds(0, CH)], d0, sd0)
    pltpu.async_copy(h_hbm.at[s1.at[pl.ds(0, GB)]], r1, sg1)

    def scan(p, c):
        pltpu.make_async_copy(nbr_hbm.at[pl.ds(c * CH, CH)], dstc[p],
                              sd[p]).wait()

        @pl.when(c + 1 < NCH)
        def _():
            pltpu.async_copy(nbr_hbm.at[pl.ds((c + 1) * CH, CH)],
                             dstc[1 - p], sd[1 - p])

        lane = lax.iota(jnp.int32, 16)

        def scan_body(q, pos):
            dls, srcs, msks, cums = [], [], [], []
            for u in range(4):
                g16 = (q * 4 + u) * 16
                dv = dstc[p][pl.ds(g16, 16)]
                eid = c * CH + g16 + lane
                srcs.append(lax.shift_right_logical(eid, 4))
                dl = dv - lo
                msk = (dl >= 0) & (dl < TPW)
                dls.append(dl)
                msks.append(msk)
                cums.append(plsc.cumsum(jnp.where(msk, 1, 0)))
            off = pos
            for u in range(4):
                idx = jnp.where(msks[u], off + cums[u] - 1, CH + lane)
                plsc.store_scatter(cdst[p], [idx], dls[u])
                plsc.store_scatter(csrc[p], [idx], srcs[u])
                off = off + cums[u][15]
            return off

        mcnt = lax.fori_loop(0, CH // 64, scan_body, 0)
        pltpu.async_copy(h_hbm.at[csrc[p].at[pl.ds(0, GB)]], rows[p], sg[p])
        return mcnt

    def fold(p, mcnt):
        lane = lax.iota(jnp.int32, 16)
        nsub = jnp.maximum((mcnt + GB - 1) // GB, 1)

        def sub_body(sb, _):
            sbase = sb * GB

            @pl.when(sb > 0)
            def _():
                pltpu.async_copy(h_hbm.at[csrc[p].at[pl.ds(sbase, GB)]],
                                 rows[p], sg[p])

            pltpu.make_async_copy(h_hbm.at[csrc[p].at[pl.ds(sbase, GB)]],
                                  rows[p], sg[p]).wait()
            nrem = jnp.maximum(jnp.minimum(mcnt - sbase, GB), 0)

            def fold_g(g, _):
                gb = g * 16
                dlv = cdst[p][pl.ds(sbase + gb, 16)]
                valid = (gb + lane) < nrem
                jv = jnp.where(valid, dlv, TPW)
                for l in range(16):
                    j = jv[l]
                    for r in range(D // 16):
                        slot = j * D + r * 16
                        v = rows[p][gb + l, pl.ds(r * 16, 16)]
                        sloc[pl.ds(slot, 16)] = jnp.maximum(
                            sloc[pl.ds(slot, 16)], v)
                return 0

            lax.fori_loop(0, (nrem + 15) // 16, fold_g, 0)
            return 0

        lax.fori_loop(0, nsub, sub_body, 0)

    def body(i, mprev):
        a = 2 * i
        ma = scan(0, a)
        fold(1, mprev)
        mb = scan(1, a + 1)
        fold(0, ma)
        return mb

    mlast = lax.fori_loop(0, NCH // 2, body, 0)
    fold(1, mlast)
    pltpu.sync_copy(sloc.at[pl.ds(0, TPW * D)],
                    out_hbm.at[pl.ds(lo * D, TPW * D)])


def _scatter_call(nbr_flat, h):
    mesh = plsc.VectorSubcoreMesh(core_axis_name="c", subcore_axis_name="s")
    fn = functools.partial(
        pl.kernel, _scatter_body, mesh=mesh,
        compiler_params=pltpu.CompilerParams(needs_layout_passes=False),
        out_type=jax.ShapeDtypeStruct((NROW * D,), jnp.float32),
        scratch_types=[
            pltpu.VMEM((CH,), jnp.int32),
            pltpu.VMEM((CH,), jnp.int32),
            pltpu.VMEM((CH + 16,), jnp.int32),
            pltpu.VMEM((CH + 16,), jnp.int32),
            pltpu.VMEM((CH + 16,), jnp.int32),
            pltpu.VMEM((CH + 16,), jnp.int32),
            pltpu.VMEM((GB, D), jnp.float32),
            pltpu.VMEM((GB, D), jnp.float32),
            pltpu.VMEM(((TPW + 1) * D,), jnp.float32),
            pltpu.SemaphoreType.DMA,
            pltpu.SemaphoreType.DMA,
            pltpu.SemaphoreType.DMA,
            pltpu.SemaphoreType.DMA,
        ],
    )()
    return fn(nbr_flat, h)


RB = 1000
NRB = N // RB


def _ln(v, g, b):
    mu = jnp.mean(v, axis=-1, keepdims=True)
    var = jnp.mean((v - mu) ** 2, axis=-1, keepdims=True)
    return (v - mu) / jnp.sqrt(var + 1e-5) * g + b


def _tail_body(x_ref, h_ref, s_ref, wu_ref, wub_ref, wo_ref, wob_ref,
               g1_ref, b1_ref, f1w_ref, f1b_ref, f2w_ref, f2b_ref,
               g2_ref, b2_ref, out_ref):
    h = h_ref[...]
    s = s_ref[...]
    md = jnp.where(s > -1.0e38, s - h, 0.0)
    wu = wu_ref[...]
    u = (jnp.dot(h, wu[:D], preferred_element_type=jnp.float32)
         + jnp.dot(md, wu[D:], preferred_element_type=jnp.float32)
         + wub_ref[...])
    h2 = jnp.dot(u, wo_ref[...], preferred_element_type=jnp.float32) + wob_ref[...]
    x1 = _ln(x_ref[...] + h2, g1_ref[...], b1_ref[...])
    z = jnp.dot(x1, f1w_ref[...], preferred_element_type=jnp.float32) + f1b_ref[...]
    f = 0.5 * z * (1.0 + lax.erf(z * 0.7071067811865476))
    f = jnp.dot(f, f2w_ref[...], preferred_element_type=jnp.float32) + f2b_ref[...]
    out_ref[...] = _ln(x1 + f, g2_ref[...], b2_ref[...])


def _tail_call(x, h, s, wu, wub, wo, wob, g1, b1, f1w, f1b, f2w, f2b, g2, b2):
    row = lambda b: (b, 0)
    full = lambda b: (0, 0)
    return pl.pallas_call(
        _tail_body,
        grid=(NRB,),
        in_specs=[
            pl.BlockSpec((RB, D), row),
            pl.BlockSpec((RB, D), row),
            pl.BlockSpec((RB, D), row),
            pl.BlockSpec((2 * D, D), full),
            pl.BlockSpec((1, D), full),
            pl.BlockSpec((D, D), full),
            pl.BlockSpec((1, D), full),
            pl.BlockSpec((1, D), full),
            pl.BlockSpec((1, D), full),
            pl.BlockSpec((D, 4 * D), full),
            pl.BlockSpec((1, 4 * D), full),
            pl.BlockSpec((4 * D, D), full),
            pl.BlockSpec((1, D), full),
            pl.BlockSpec((1, D), full),
            pl.BlockSpec((1, D), full),
        ],
        out_specs=pl.BlockSpec((RB, D), row),
        out_shape=jax.ShapeDtypeStruct((N, D), jnp.float32),
    )(x, h, s, wu, wub, wo, wob, g1, b1, f1w, f1b, f2w, f2b, g2, b2)


def kernel(x, W_in_w, W_in_b, Wu_w, Wu_b, W_out_w, W_out_b, ln1_g, ln1_b,
           ffn1_w, ffn1_b, ffn2_w, ffn2_b, ln2_g, ln2_b):
    xpad = jnp.concatenate([x, jnp.zeros((NPAD - N, D), jnp.float32)], axis=0)
    sq = jnp.sum(xpad * xpad, axis=1).reshape(1, NPAD)
    nbr, h = _knn_call(x, xpad, sq, W_in_w, W_in_b.reshape(1, D))
    s_flat = _scatter_call(nbr.reshape(-1), h)
    s = s_flat.reshape(NROW, D)[:N]
    return _tail_call(x, h, s, Wu_w, Wu_b.reshape(1, D), W_out_w,
                      W_out_b.reshape(1, D), ln1_g.reshape(1, D),
                      ln1_b.reshape(1, D), ffn1_w, ffn1_b.reshape(1, 4 * D),
                      ffn2_w, ffn2_b.reshape(1, D), ln2_g.reshape(1, D),
                      ln2_b.reshape(1, D))


# speedup vs baseline: 3.7816x; 1.0007x over previous
import functools

import jax
import jax.numpy as jnp
from jax import lax
from jax.experimental import pallas as pl
from jax.experimental.pallas import tpu as pltpu
from jax.experimental.pallas import tpu_sc as plsc

N = 10000
D = 128
K = 16
NPAD = 10240
QB = 200
NBLK = N // QB

NW = 32
TPW = 313
NROW = NW * TPW
E = N * K
CH = 3200
NCH = E // CH
GB = 128
SENT = -3.0e38


def _knn_body(xq_ref, xall_ref, sq_ref, ww_ref, wb_ref, nbr_ref, h_ref):
    b = pl.program_id(0)
    xq = xq_ref[...]
    p = lax.dot_general(xq, xall_ref[...], (((1,), (1,)), ((), ())),
                        preferred_element_type=jnp.float32)
    d = sq_ref[...] - 2.0 * p
    gi = b * QB + lax.broadcasted_iota(jnp.int32, (QB, 1), 0)
    col = lax.broadcasted_iota(jnp.int32, (QB, NPAD), 1)
    d = jnp.where((col == gi) | (col >= N), jnp.inf, d)
    idxs = []
    for _ in range(K):
        m = jnp.min(d, axis=1, keepdims=True)
        isel = jnp.min(jnp.where(d == m, col, NPAD), axis=1, keepdims=True)
        idxs.append(isel)
        d = jnp.where(col == isel, jnp.inf, d)
    nbr_ref[...] = jnp.concatenate(idxs, axis=1)
    h_ref[...] = (jnp.dot(xq, ww_ref[...], preferred_element_type=jnp.float32)
                  + wb_ref[...])


def _knn_call(x, xpad, sq, ww, wb):
    return pl.pallas_call(
        _knn_body,
        grid=(NBLK,),
        in_specs=[
            pl.BlockSpec((QB, D), lambda b: (b, 0)),
            pl.BlockSpec((NPAD, D), lambda b: (0, 0)),
            pl.BlockSpec((1, NPAD), lambda b: (0, 0)),
            pl.BlockSpec((D, D), lambda b: (0, 0)),
            pl.BlockSpec((1, D), lambda b: (0, 0)),
        ],
        out_specs=[
            pl.BlockSpec((QB, K), lambda b: (b, 0)),
            pl.BlockSpec((QB, D), lambda b: (b, 0)),
        ],
        out_shape=[
            jax.ShapeDtypeStruct((N, K), jnp.int32),
            jax.ShapeDtypeStruct((N, D), jnp.float32),
        ],
    )(x, xpad, sq, ww, wb)


def _scatter_body(nbr_hbm, h_hbm, out_hbm, d0, d1, c0, c1, s0, s1, r0, r1,
                  sloc, sd0, sd1, sg0, sg1):
    dstc = (d0, d1)
    cdst = (c0, c1)
    csrc = (s0, s1)
    rows = (r0, r1)
    sd = (sd0, sd1)
    sg = (sg0, sg1)
    wid = lax.axis_index("s") * 2 + lax.axis_index("c")
    lo = wid * TPW

    def init_body(i, _):
        sloc[pl.ds(i * 16, 16)] = jnp.full((16,), SENT, jnp.float32)
        return 0

    lax.fori_loop(0, (TPW + 1) * D // 16, init_body, 0)

    def init_idx(i, _):
        z = jnp.zeros((16,), jnp.int32)
        c0[pl.ds(i * 16, 16)] = z
        c1[pl.ds(i * 16, 16)] = z
        s0[pl.ds(i * 16, 16)] = z
        s1[pl.ds(i * 16, 16)] = z
        return 0

    lax.fori_loop(0, (CH + 16) // 16, init_idx, 0)

    pltpu.async_copy(nbr_hbm.at[pl.ds(0, CH)], d0, sd0)
    pltpu.async_copy(h_hbm.at[s1.at[pl.ds(0, GB)]], r1, sg1)

    def scan(p, c):
        pltpu.make_async_copy(nbr_hbm.at[pl.ds(c * CH, CH)], dstc[p],
                              sd[p]).wait()

        @pl.when(c + 1 < NCH)
        def _():
            pltpu.async_copy(nbr_hbm.at[pl.ds((c + 1) * CH, CH)],
                             dstc[1 - p], sd[1 - p])

        lane = lax.iota(jnp.int32, 16)

        def scan_body(q, pos):
            dls, srcs, msks, cums = [], [], [], []
            for u in range(4):
                g16 = (q * 4 + u) * 16
                dv = dstc[p][pl.ds(g16, 16)]
                eid = c * CH + g16 + lane
                srcs.append(lax.shift_right_logical(eid, 4))
                dl = dv - lo
                msk = (dl >= 0) & (dl < TPW)
                dls.append(dl)
                msks.append(msk)
                cums.append(plsc.cumsum(jnp.where(msk, 1, 0)))
            off = pos
            for u in range(4):
                idx = jnp.where(msks[u], off + cums[u] - 1, CH + lane)
                plsc.store_scatter(cdst[p], [idx], dls[u])
                plsc.store_scatter(csrc[p], [idx], srcs[u])
                off = off + cums[u][15]
            return off

        mcnt = lax.fori_loop(0, CH // 64, scan_body, 0)
        pltpu.async_copy(h_hbm.at[csrc[p].at[pl.ds(0, GB)]], rows[p], sg[p])
        return mcnt

    def fold(p, mcnt):
        lane = lax.iota(jnp.int32, 16)
        nsub = jnp.maximum((mcnt + GB - 1) // GB, 1)

        def sub_body(sb, _):
            sbase = sb * GB

            @pl.when(sb > 0)
            def _():
                pltpu.async_copy(h_hbm.at[csrc[p].at[pl.ds(sbase, GB)]],
                                 rows[p], sg[p])

            pltpu.make_async_copy(h_hbm.at[csrc[p].at[pl.ds(sbase, GB)]],
                                  rows[p], sg[p]).wait()
            nrem = jnp.maximum(jnp.minimum(mcnt - sbase, GB), 0)

            def fold_g(g, _):
                gb = g * 16
                dlv = cdst[p][pl.ds(sbase + gb, 16)]
                valid = (gb + lane) < nrem
                jv = jnp.where(valid, dlv, TPW)
                for l in range(16):
                    j = jv[l]
                    for r in range(D // 16):
                        slot = j * D + r * 16
                        v = rows[p][gb + l, pl.ds(r * 16, 16)]
                        sloc[pl.ds(slot, 16)] = jnp.maximum(
                            sloc[pl.ds(slot, 16)], v)
                return 0

            lax.fori_loop(0, (nrem + 15) // 16, fold_g, 0)
            return 0

        lax.fori_loop(0, nsub, sub_body, 0)

    def body(i, mprev):
        a = 2 * i
        ma = scan(0, a)
        fold(1, mprev)
        mb = scan(1, a + 1)
        fold(0, ma)
        return mb

    mlast = lax.fori_loop(0, NCH // 2, body, 0)
    fold(1, mlast)
    pltpu.sync_copy(sloc.at[pl.ds(0, TPW * D)],
                    out_hbm.at[pl.ds(lo * D, TPW * D)])


def _scatter_call(nbr_flat, h):
    mesh = plsc.VectorSubcoreMesh(core_axis_name="c", subcore_axis_name="s")
    fn = functools.partial(
        pl.kernel, _scatter_body, mesh=mesh,
        compiler_params=pltpu.CompilerParams(needs_layout_passes=False),
        out_type=jax.ShapeDtypeStruct((NROW * D,), jnp.float32),
        scratch_types=[
            pltpu.VMEM((CH,), jnp.int32),
            pltpu.VMEM((CH,), jnp.int32),
            pltpu.VMEM((CH + 16,), jnp.int32),
            pltpu.VMEM((CH + 16,), jnp.int32),
            pltpu.VMEM((CH + 16,), jnp.int32),
            pltpu.VMEM((CH + 16,), jnp.int32),
            pltpu.VMEM((GB, D), jnp.float32),
            pltpu.VMEM((GB, D), jnp.float32),
            pltpu.VMEM(((TPW + 1) * D,), jnp.float32),
            pltpu.SemaphoreType.DMA,
            pltpu.SemaphoreType.DMA,
            pltpu.SemaphoreType.DMA,
            pltpu.SemaphoreType.DMA,
        ],
    )()
    return fn(nbr_flat, h)


RB = 1000
NRB = N // RB


def _ln(v, g, b):
    mu = jnp.mean(v, axis=-1, keepdims=True)
    var = jnp.mean((v - mu) ** 2, axis=-1, keepdims=True)
    return (v - mu) / jnp.sqrt(var + 1e-5) * g + b


def _tail_body(x_ref, h_ref, s_ref, wu_ref, wub_ref, wo_ref, wob_ref,
               g1_ref, b1_ref, f1w_ref, f1b_ref, f2w_ref, f2b_ref,
               g2_ref, b2_ref, out_ref):
    h = h_ref[...]
    s = s_ref[...]
    md = jnp.where(s > -1.0e38, s - h, 0.0)
    wu = wu_ref[...]
    u = (jnp.dot(h, wu[:D], preferred_element_type=jnp.float32)
         + jnp.dot(md, wu[D:], preferred_element_type=jnp.float32)
         + wub_ref[...])
    h2 = jnp.dot(u, wo_ref[...], preferred_element_type=jnp.float32) + wob_ref[...]
    x1 = _ln(x_ref[...] + h2, g1_ref[...], b1_ref[...])
    z = jnp.dot(x1, f1w_ref[...], preferred_element_type=jnp.float32) + f1b_ref[...]
    f = 0.5 * z * (1.0 + lax.erf(z * 0.7071067811865476))
    f = jnp.dot(f, f2w_ref[...], preferred_element_type=jnp.float32) + f2b_ref[...]
    out_ref[...] = _ln(x1 + f, g2_ref[...], b2_ref[...])


def _tail_call(x, h, s, wu, wub, wo, wob, g1, b1, f1w, f1b, f2w, f2b, g2, b2):
    row = lambda b: (b, 0)
    full = lambda b: (0, 0)
    return pl.pallas_call(
        _tail_body,
        grid=(NRB,),
        in_specs=[
            pl.BlockSpec((RB, D), row),
            pl.BlockSpec((RB, D), row),
            pl.BlockSpec((RB, D), row),
            pl.BlockSpec((2 * D, D), full),
            pl.BlockSpec((1, D), full),
            pl.BlockSpec((D, D), full),
            pl.BlockSpec((1, D), full),
            pl.BlockSpec((1, D), full),
            pl.BlockSpec((1, D), full),
            pl.BlockSpec((D, 4 * D), full),
            pl.BlockSpec((1, 4 * D), full),
            pl.BlockSpec((4 * D, D), full),
            pl.BlockSpec((1, D), full),
            pl.BlockSpec((1, D), full),
            pl.BlockSpec((1, D), full),
        ],
        out_specs=pl.BlockSpec((RB, D), row),
        out_shape=jax.ShapeDtypeStruct((N, D), jnp.float32),
    )(x, h, s, wu, wub, wo, wob, g1, b1, f1w, f1b, f2w, f2b, g2, b2)


def kernel(x, W_in_w, W_in_b, Wu_w, Wu_b, W_out_w, W_out_b, ln1_g, ln1_b,
           ffn1_w, ffn1_b, ffn2_w, ffn2_b, ln2_g, ln2_b):
    xpad = jnp.concatenate([x, jnp.zeros((NPAD - N, D), jnp.float32)], axis=0)
    sq = jnp.sum(xpad * xpad, axis=1).reshape(1, NPAD)
    nbr, h = _knn_call(x, xpad, sq, W_in_w, W_in_b.reshape(1, D))
    s_flat = _scatter_call(nbr.reshape(-1), h)
    s = s_flat.reshape(NROW, D)[:N]
    return _tail_call(x, h, s, Wu_w, Wu_b.reshape(1, D), W_out_w,
                      W_out_b.reshape(1, D), ln1_g.reshape(1, D),
                      ln1_b.reshape(1, D), ffn1_w, ffn1_b.reshape(1, 4 * D),
                      ffn2_w, ffn2_b.reshape(1, D), ln2_g.reshape(1, D),
                      ln2_b.reshape(1, D))
